# Initial kernel scaffold; baseline (speedup 1.0000x reference)
import jax
import jax.numpy as jnp
from jax.experimental import pallas as pl


def kernel(x0, x1, x2, ei0, ei1, ei2, W_fc1, b_fc1, Wc1, bc1, Wc2, bc2, Wd1, bd1, Wd2, bd2, W_fc2, b_fc2, Wmix):
    raise NotImplementedError("write your pallas kernel here")


# speedup vs baseline: 7.4964x; 7.4964x over previous
import functools

import jax
import jax.numpy as jnp
from jax import lax
from jax.experimental import pallas as pl
from jax.experimental.pallas import tpu as pltpu
from jax.experimental.pallas import tpu_sc as plsc

N = 10000
NP = 10240
D = 128
G = 3
E = 320000
E_PAD = 323584
N_PAD_ROWS = NP - N
ROWS_PER_TILE = NP // 16
NB = NP // 128
CHUNK = 128

_mesh = plsc.VectorSubcoreMesh(core_axis_name="c", subcore_axis_name="s")


def _deg_body(dst_hbm, init_hbm, dega_hbm, degb_hbm, acc, idx_v, ones_v, sem):
    c = lax.axis_index("c")
    s = lax.axis_index("s")
    for j in range(8):
        ones_v[pl.ds(16 * j, 16)] = jnp.full((16,), 1.0, jnp.float32)
    per_worker = E_PAD // 32
    n_chunks = per_worker // CHUNK
    row0 = s * ROWS_PER_TILE

    for g in range(G):
        pltpu.sync_copy(init_hbm.at[c].at[0, pl.ds(row0, ROWS_PER_TILE)],
                        acc.at[pl.ds(row0, ROWS_PER_TILE)])
        plsc.subcore_barrier()

        def body(gi, carry, g=g):
            base = gi * CHUNK
            pltpu.sync_copy(dst_hbm.at[g].at[0, pl.ds(base, CHUNK)], idx_v)
            pltpu.sync_copy(ones_v, acc.at[idx_v], add=True)
            return carry

        @pl.when(c == 0)
        def _():
            base0 = s * per_worker
            lax.fori_loop(base0 // CHUNK, base0 // CHUNK + n_chunks, body, 0)

        @pl.when(c == 1)
        def _():
            base1 = E_PAD // 2 + s * per_worker
            lax.fori_loop(base1 // CHUNK, base1 // CHUNK + n_chunks, body, 0)

        plsc.subcore_barrier()

        @pl.when(c == 0)
        def _():
            pltpu.sync_copy(acc.at[pl.ds(row0, ROWS_PER_TILE)],
                            dega_hbm.at[g].at[0, pl.ds(row0, ROWS_PER_TILE)])

        @pl.when(c == 1)
        def _():
            pltpu.sync_copy(acc.at[pl.ds(row0, ROWS_PER_TILE)],
                            degb_hbm.at[g].at[0, pl.ds(row0, ROWS_PER_TILE)])

        plsc.subcore_barrier()


def _deg_call(dst3, init2):
    f = pl.kernel(
        _deg_body,
        out_type=[jax.ShapeDtypeStruct((G, 1, NP), jnp.float32),
                  jax.ShapeDtypeStruct((G, 1, NP), jnp.float32)],
        mesh=_mesh,
        scratch_types=[
            pltpu.VMEM_SHARED((NP,), jnp.float32),
            pltpu.VMEM((CHUNK,), jnp.int32),
            pltpu.VMEM((CHUNK,), jnp.float32),
            pltpu.SemaphoreType.DMA,
        ],
    )
    return f(dst3, init2)


def _prop_body(enc, za_hbm, zb_hbm, src_hbm, dst_hbm, outa_hbm, outb_hbm,
               acc, sidx, didx, rows, sem):
    c = lax.axis_index("c")
    s = lax.axis_index("s")
    row0 = s * ROWS_PER_TILE
    if enc:
        per_worker = E_PAD // 32
    else:
        per_worker = E_PAD // 16
    n_chunks = per_worker // CHUNK

    for g in range(G):
        @pl.when(c == 0)
        def _():
            pltpu.sync_copy(za_hbm.at[g].at[pl.ds(row0, ROWS_PER_TILE)],
                            acc.at[pl.ds(row0, ROWS_PER_TILE)])

        @pl.when(c == 1)
        def _():
            pltpu.sync_copy(zb_hbm.at[g].at[pl.ds(row0, ROWS_PER_TILE)],
                            acc.at[pl.ds(row0, ROWS_PER_TILE)])

        plsc.subcore_barrier()

        def body(gi, carry, g=g, src_z=None):
            base = gi * CHUNK
            pltpu.sync_copy(src_hbm.at[g].at[0, pl.ds(base, CHUNK)], sidx)
            pltpu.sync_copy(dst_hbm.at[g].at[0, pl.ds(base, CHUNK)], didx)
            pltpu.async_copy(src_z.at[g].at[sidx], rows, sem).wait()
            pltpu.sync_copy(rows, acc.at[didx], add=True)
            return carry

        if enc:
            @pl.when(c == 0)
            def _():
                b0 = (s * per_worker) // CHUNK
                lax.fori_loop(b0, b0 + n_chunks,
                              functools.partial(body, src_z=za_hbm), 0)

            @pl.when(c == 1)
            def _():
                b1 = (E_PAD // 2 + s * per_worker) // CHUNK
                lax.fori_loop(b1, b1 + n_chunks,
                              functools.partial(body, src_z=za_hbm), 0)
        else:
            @pl.when(c == 0)
            def _():
                b0 = (s * per_worker) // CHUNK
                lax.fori_loop(b0, b0 + n_chunks,
                              functools.partial(body, src_z=za_hbm), 0)

            @pl.when(c == 1)
            def _():
                b1 = (s * per_worker) // CHUNK
                lax.fori_loop(b1, b1 + n_chunks,
                              functools.partial(body, src_z=zb_hbm), 0)

        plsc.subcore_barrier()

        @pl.when(c == 0)
        def _():
            pltpu.sync_copy(acc.at[pl.ds(row0, ROWS_PER_TILE)],
                            outa_hbm.at[g].at[pl.ds(row0, ROWS_PER_TILE)])

        @pl.when(c == 1)
        def _():
            pltpu.sync_copy(acc.at[pl.ds(row0, ROWS_PER_TILE)],
                            outb_hbm.at[g].at[pl.ds(row0, ROWS_PER_TILE)])

        plsc.subcore_barrier()


def _prop_call(enc, za, zb, src3, dst3):
    f = pl.kernel(
        functools.partial(_prop_body, enc),
        out_type=[jax.ShapeDtypeStruct((G, NP, D), jnp.float32),
                  jax.ShapeDtypeStruct((G, NP, D), jnp.float32)],
        mesh=_mesh,
        scratch_types=[
            pltpu.VMEM_SHARED((NP, D), jnp.float32),
            pltpu.VMEM((CHUNK,), jnp.int32),
            pltpu.VMEM((CHUNK,), jnp.int32),
            pltpu.VMEM((CHUNK, D), jnp.float32),
            pltpu.SemaphoreType.DMA,
        ],
    )
    return f(za, zb, src3, dst3)


def _row_mask(nb):
    rows = nb * 128 + lax.broadcasted_iota(jnp.int32, (128, D), 0)
    return rows < N


def _t0_body(x_ref, w1_ref, b1_ref, wc1_ref, dinv_ref, pre_ref, z1_ref):
    xb = x_ref[0]
    pre = jnp.dot(xb, w1_ref[0], preferred_element_type=jnp.float32) + b1_ref[0]
    pre_ref[0] = pre
    z1_ref[0] = jnp.dot(pre, wc1_ref[...],
                        preferred_element_type=jnp.float32) * dinv_ref[0]


def _t0(xs, W_fc1, b_fc1, Wc1, dinv_b):
    return pl.pallas_call(
        _t0_body,
        grid=(G, NB),
        in_specs=[
            pl.BlockSpec((1, 128, D), lambda i, nb: (i, nb, 0)),
            pl.BlockSpec((1, D, D), lambda i, nb: (i, 0, 0)),
            pl.BlockSpec((1, 1, D), lambda i, nb: (i, 0, 0)),
            pl.BlockSpec((D, D), lambda i, nb: (0, 0)),
            pl.BlockSpec((1, 128, D), lambda i, nb: (i, nb, 0)),
        ],
        out_specs=[
            pl.BlockSpec((1, 128, D), lambda i, nb: (i, nb, 0)),
            pl.BlockSpec((1, 128, D), lambda i, nb: (i, nb, 0)),
        ],
        out_shape=[jax.ShapeDtypeStruct((G, NP, D), jnp.float32),
                   jax.ShapeDtypeStruct((G, NP, D), jnp.float32)],
    )(xs, W_fc1, b_fc1.reshape(G, 1, D), Wc1, dinv_b)


def _t1_body(ra_ref, rb_ref, dinv_ref, b_ref, w_ref, h_ref, z_ref):
    dinv = dinv_ref[0]
    h = (ra_ref[0] + rb_ref[0]) * dinv + b_ref[0]
    h_ref[0] = h
    z_ref[0] = jnp.dot(h, w_ref[...], preferred_element_type=jnp.float32) * dinv


def _t1(ra, rb, dinv_b, b, w):
    return pl.pallas_call(
        _t1_body,
        grid=(G, NB),
        in_specs=[
            pl.BlockSpec((1, 128, D), lambda i, nb: (i, nb, 0)),
            pl.BlockSpec((1, 128, D), lambda i, nb: (i, nb, 0)),
            pl.BlockSpec((1, 128, D), lambda i, nb: (i, nb, 0)),
            pl.BlockSpec((1, 1, D), lambda i, nb: (0, 0, 0)),
            pl.BlockSpec((D, D), lambda i, nb: (0, 0)),
        ],
        out_specs=[
            pl.BlockSpec((1, 128, D), lambda i, nb: (i, nb, 0)),
            pl.BlockSpec((1, 128, D), lambda i, nb: (i, nb, 0)),
        ],
        out_shape=[jax.ShapeDtypeStruct((G, NP, D), jnp.float32),
                   jax.ShapeDtypeStruct((G, NP, D), jnp.float32)],
    )(ra, rb, dinv_b, b.reshape(1, 1, D), w)


def _t2_body(ra_ref, rb_ref, dinv_ref, b_ref, w_ref,
             enc_ref, za_ref, zb_ref, gs_ref):
    nb = pl.program_id(0)
    mask = _row_mask(nb)
    encs = []
    for i in range(G):
        enc_i = (ra_ref[i] + rb_ref[i]) * dinv_ref[i] + b_ref[0]
        encs.append(enc_i)
    tot = encs[0] + encs[1] + encs[2]
    for i in range(G):
        enc_ref[i] = encs[i]
        dinv = dinv_ref[i]
        za_ref[i] = jnp.dot(encs[i], w_ref[...],
                            preferred_element_type=jnp.float32) * dinv
        dec_i = (tot - encs[i]) * 0.5
        zb_ref[i] = jnp.dot(dec_i, w_ref[...],
                            preferred_element_type=jnp.float32) * dinv
        gs_ref[i, 0] = jnp.sum(jnp.where(mask, encs[i], 0.0), axis=0,
                               keepdims=True)


def _t2(ra, rb, dinv_b, b, w):
    return pl.pallas_call(
        _t2_body,
        grid=(NB,),
        in_specs=[
            pl.BlockSpec((G, 128, D), lambda nb: (0, nb, 0)),
            pl.BlockSpec((G, 128, D), lambda nb: (0, nb, 0)),
            pl.BlockSpec((G, 128, D), lambda nb: (0, nb, 0)),
            pl.BlockSpec((1, 1, D), lambda nb: (0, 0, 0)),
            pl.BlockSpec((D, D), lambda nb: (0, 0)),
        ],
        out_specs=[
            pl.BlockSpec((G, 128, D), lambda nb: (0, nb, 0)),
            pl.BlockSpec((G, 128, D), lambda nb: (0, nb, 0)),
            pl.BlockSpec((G, 128, D), lambda nb: (0, nb, 0)),
            pl.BlockSpec((G, 1, 1, D), lambda nb: (0, nb, 0, 0)),
        ],
        out_shape=[jax.ShapeDtypeStruct((G, NP, D), jnp.float32),
                   jax.ShapeDtypeStruct((G, NP, D), jnp.float32),
                   jax.ShapeDtypeStruct((G, NP, D), jnp.float32),
                   jax.ShapeDtypeStruct((G, NB, 1, D), jnp.float32)],
    )(ra, rb, dinv_b, b.reshape(1, 1, D), w)


def _t3_body(ra_ref, rb_ref, conv1_ref, dinv_ref, b_ref, w_ref,
             za2_ref, zb2_ref, r1_ref):
    nb = pl.program_id(1)
    mask = _row_mask(nb)
    dinv = dinv_ref[0]
    a = ra_ref[0] * dinv + b_ref[0]
    b1 = rb_ref[0] * dinv + b_ref[0]
    za2_ref[0] = jnp.dot(a, w_ref[...], preferred_element_type=jnp.float32) * dinv
    zb2_ref[0] = jnp.dot(b1, w_ref[...], preferred_element_type=jnp.float32) * dinv
    dlt = jnp.where(mask, conv1_ref[0] - a, 0.0)
    r1_ref[0, 0] = jnp.sum(dlt * dlt, axis=0, keepdims=True)


def _t3(ra, rb, conv1, dinv_b, b, w):
    return pl.pallas_call(
        _t3_body,
        grid=(G, NB),
        in_specs=[
            pl.BlockSpec((1, 128, D), lambda i, nb: (i, nb, 0)),
            pl.BlockSpec((1, 128, D), lambda i, nb: (i, nb, 0)),
            pl.BlockSpec((1, 128, D), lambda i, nb: (i, nb, 0)),
            pl.BlockSpec((1, 128, D), lambda i, nb: (i, nb, 0)),
            pl.BlockSpec((1, 1, D), lambda i, nb: (0, 0, 0)),
            pl.BlockSpec((D, D), lambda i, nb: (0, 0)),
        ],
        out_specs=[
            pl.BlockSpec((1, 128, D), lambda i, nb: (i, nb, 0)),
            pl.BlockSpec((1, 128, D), lambda i, nb: (i, nb, 0)),
            pl.BlockSpec((1, 1, 1, D), lambda i, nb: (i, nb, 0, 0)),
        ],
        out_shape=[jax.ShapeDtypeStruct((G, NP, D), jnp.float32),
                   jax.ShapeDtypeStruct((G, NP, D), jnp.float32),
                   jax.ShapeDtypeStruct((G, NB, 1, D), jnp.float32)],
    )(ra, rb, conv1, dinv_b, b.reshape(1, 1, D), w)


def _t4_body(ra_ref, rb_ref, pre_ref, dinv_ref, b_ref, w2a_ref, w2b_ref,
             bf_ref, fin_ref, r2_ref):
    nb = pl.program_id(1)
    mask = _row_mask(nb)
    dinv = dinv_ref[0]
    a2 = ra_ref[0] * dinv + b_ref[0]
    b2 = rb_ref[0] * dinv + b_ref[0]
    fin_ref[0] = (jnp.dot(a2, w2a_ref[...], preferred_element_type=jnp.float32)
                  + jnp.dot(b2, w2b_ref[...], preferred_element_type=jnp.float32)
                  + bf_ref[0])
    dlt = jnp.where(mask, pre_ref[0] - a2, 0.0)
    r2_ref[0, 0] = jnp.sum(dlt * dlt, axis=0, keepdims=True)


def _t4(ra, rb, pre, dinv_b, b, w2a, w2b, bf):
    return pl.pallas_call(
        _t4_body,
        grid=(G, NB),
        in_specs=[
            pl.BlockSpec((1, 128, D), lambda i, nb: (i, nb, 0)),
            pl.BlockSpec((1, 128, D), lambda i, nb: (i, nb, 0)),
            pl.BlockSpec((1, 128, D), lambda i, nb: (i, nb, 0)),
            pl.BlockSpec((1, 128, D), lambda i, nb: (i, nb, 0)),
            pl.BlockSpec((1, 1, D), lambda i, nb: (0, 0, 0)),
            pl.BlockSpec((D, D), lambda i, nb: (0, 0)),
            pl.BlockSpec((D, D), lambda i, nb: (0, 0)),
            pl.BlockSpec((1, 1, D), lambda i, nb: (0, 0, 0)),
        ],
        out_specs=[
            pl.BlockSpec((1, 128, D), lambda i, nb: (i, nb, 0)),
            pl.BlockSpec((1, 1, 1, D), lambda i, nb: (i, nb, 0, 0)),
        ],
        out_shape=[jax.ShapeDtypeStruct((G, NP, D), jnp.float32),
                   jax.ShapeDtypeStruct((G, NB, 1, D), jnp.float32)],
    )(ra, rb, pre, dinv_b, b.reshape(1, 1, D), w2a, w2b, bf.reshape(1, 1, D))


def _t5_body(fin_ref, enc_ref, wmix_ref, gv_ref,
             fuse_ref, used_ref, comp_ref, o0_ref):
    nb = pl.program_id(0)
    mask = _row_mask(nb)
    fins = [fin_ref[i] for i in range(G)]
    encs = [enc_ref[i] for i in range(G)]
    fsum = (fins[0] + fins[1] + fins[2]) * (1.0 / 3.0)
    fuse_ref[...] = jax.nn.sigmoid(fsum)
    esum = (encs[0] + encs[1] + encs[2]) * (1.0 / 3.0)
    used_ref[...] = jnp.where(esum > 0, esum,
                              jnp.exp(jnp.minimum(esum, 0.0)) - 1.0)
    cols = []
    for i in range(G):
        ci = []
        for j in range(G):
            sij = jnp.dot(encs[j], wmix_ref[i],
                          preferred_element_type=jnp.float32)
            tij = jnp.sum(sij * gv_ref[i], axis=1, keepdims=True)
            ci.append(jax.nn.sigmoid(tij))
        den = ci[0] + ci[1] + ci[2]
        cols.extend([c / den for c in ci])
    comp_ref[...] = jnp.concatenate(
        cols + [jnp.zeros((128, D - 3 * G), jnp.float32)], axis=1)
    pairs = [(0, 1), (0, 2), (1, 2)]
    for p, (i, j) in enumerate(pairs):
        dlt = jnp.where(mask, fins[i] - fins[j], 0.0)
        o0_ref[p, 0] = jnp.sum(dlt * dlt, axis=0, keepdims=True)


def _t5(fin, enc, Wmix, gvec):
    return pl.pallas_call(
        _t5_body,
        grid=(NB,),
        in_specs=[
            pl.BlockSpec((G, 128, D), lambda nb: (0, nb, 0)),
            pl.BlockSpec((G, 128, D), lambda nb: (0, nb, 0)),
            pl.BlockSpec((G, D, D), lambda nb: (0, 0, 0)),
            pl.BlockSpec((G, 1, D), lambda nb: (0, 0, 0)),
        ],
        out_specs=[
            pl.BlockSpec((128, D), lambda nb: (nb, 0)),
            pl.BlockSpec((128, D), lambda nb: (nb, 0)),
            pl.BlockSpec((128, D), lambda nb: (nb, 0)),
            pl.BlockSpec((G, 1, 1, D), lambda nb: (0, nb, 0, 0)),
        ],
        out_shape=[jax.ShapeDtypeStruct((NP, D), jnp.float32),
                   jax.ShapeDtypeStruct((NP, D), jnp.float32),
                   jax.ShapeDtypeStruct((NP, D), jnp.float32),
                   jax.ShapeDtypeStruct((G, NB, 1, D), jnp.float32)],
    )(fin, enc, Wmix, gvec.reshape(G, 1, D))


def kernel(x0, x1, x2, ei0, ei1, ei2, W_fc1, b_fc1, Wc1, bc1, Wc2, bc2,
           Wd1, bd1, Wd2, bd2, W_fc2, b_fc2, Wmix):
    pad = ((0, N_PAD_ROWS), (0, 0))
    xs = jnp.stack([jnp.pad(x0, pad), jnp.pad(x1, pad), jnp.pad(x2, pad)])
    pad_idx = (N + (jnp.arange(E_PAD - E, dtype=jnp.int32) % N_PAD_ROWS))
    srcs, dsts = [], []
    for ei in (ei0, ei1, ei2):
        srcs.append(jnp.concatenate([ei[0].astype(jnp.int32), pad_idx]))
        dsts.append(jnp.concatenate([ei[1].astype(jnp.int32), pad_idx]))
    src3 = jnp.stack(srcs).reshape(G, 1, E_PAD)
    dst3 = jnp.stack(dsts).reshape(G, 1, E_PAD)
    init2 = jnp.stack([jnp.ones((NP,), jnp.float32),
                       jnp.zeros((NP,), jnp.float32)]).reshape(2, 1, NP)
    zeros3 = jnp.zeros((G, NP, D), jnp.float32)

    dega, degb = _deg_call(dst3, init2)
    deg = (dega + degb).reshape(G, NP)
    dinv = lax.rsqrt(jnp.maximum(deg, 1e-12))
    dinv_b = jnp.broadcast_to(dinv[:, :, None], (G, NP, D))

    pre, z1 = _t0(xs, W_fc1, b_fc1, Wc1, dinv_b)
    r1a, r1b = _prop_call(True, z1, zeros3, src3, dst3)
    conv1, z2 = _t1(r1a, r1b, dinv_b, bc1, Wc2)
    r2a, r2b = _prop_call(True, z2, zeros3, src3, dst3)
    enc, za, zb, gsum = _t2(r2a, r2b, dinv_b, bc2, Wd1)

    r3a, r3b = _prop_call(False, za, zb, src3, dst3)
    za2, zb2, r1sum = _t3(r3a, r3b, conv1, dinv_b, bd1, Wd2)
    r4a, r4b = _prop_call(False, za2, zb2, src3, dst3)
    fin, r2sum = _t4(r4a, r4b, pre, dinv_b, bd2,
                     W_fc2[:D], W_fc2[D:], b_fc2)

    gvec = jax.nn.sigmoid(jnp.sum(gsum, axis=(1, 2)) / N)
    fuse, used, comp, o0sum = _t5(fin, enc, Wmix, gvec)

    obf1 = 0.0
    for i in range(G):
        obf1 = obf1 + (jnp.sqrt(jnp.sum(r2sum[i]))
                       + jnp.sqrt(jnp.sum(r1sum[i]))) / 2.0
    obf0 = 0.0
    for p in range(3):
        obf0 = obf0 + 2.0 * jnp.sqrt(jnp.sum(o0sum[p]))

    return (fuse[:N], used[:N], comp[:N, :3 * G], obf1, obf0)


# speedup vs baseline: 13.3355x; 1.7789x over previous
import functools

import jax
import jax.numpy as jnp
from jax import lax
from jax.experimental import pallas as pl
from jax.experimental.pallas import tpu as pltpu
from jax.experimental.pallas import tpu_sc as plsc

N = 10000
NP = 10240
D = 128
G = 3
E = 320000
E_PAD = 327680
N_PAD_ROWS = NP - N
ROWS_PER_TILE = NP // 16
NB = NP // 128
CHUNK = 128

_mesh = plsc.VectorSubcoreMesh(core_axis_name="c", subcore_axis_name="s")


def _deg_body(dst_hbm, init_hbm, dega_hbm, degb_hbm,
              acc, idx_all, ones_v, sem):
    c = lax.axis_index("c")
    s = lax.axis_index("s")
    for j in range(8):
        ones_v[pl.ds(16 * j, 16)] = jnp.full((16,), 1.0, jnp.float32)
    n_chunks = E_PAD // 32 // CHUNK
    row0 = s * ROWS_PER_TILE

    for g in range(G):
        pltpu.sync_copy(init_hbm.at[c].at[0, pl.ds(row0, ROWS_PER_TILE)],
                        acc.at[pl.ds(row0, ROWS_PER_TILE)])

        @pl.when(c == 0)
        def _():
            pltpu.sync_copy(dst_hbm.at[g].at[pl.ds(s * n_chunks, n_chunks)],
                            idx_all)

        @pl.when(c == 1)
        def _():
            pltpu.sync_copy(
                dst_hbm.at[g].at[pl.ds(E_PAD // 256 + s * n_chunks, n_chunks)],
                idx_all)

        plsc.subcore_barrier()

        def body(j, carry):
            pltpu.sync_copy(ones_v, acc.at[idx_all.at[j]], add=True)
            return carry

        lax.fori_loop(0, n_chunks, body, 0)
        plsc.subcore_barrier()

        @pl.when(c == 0)
        def _():
            pltpu.sync_copy(acc.at[pl.ds(row0, ROWS_PER_TILE)],
                            dega_hbm.at[g].at[0, pl.ds(row0, ROWS_PER_TILE)])

        @pl.when(c == 1)
        def _():
            pltpu.sync_copy(acc.at[pl.ds(row0, ROWS_PER_TILE)],
                            degb_hbm.at[g].at[0, pl.ds(row0, ROWS_PER_TILE)])

        plsc.subcore_barrier()


def _deg_call(dst3, init2):
    f = pl.kernel(
        _deg_body,
        out_type=[jax.ShapeDtypeStruct((G, 1, NP), jnp.float32),
                  jax.ShapeDtypeStruct((G, 1, NP), jnp.float32)],
        mesh=_mesh,
        scratch_types=[
            pltpu.VMEM_SHARED((NP,), jnp.float32),
            pltpu.VMEM((E_PAD // 32 // CHUNK, CHUNK), jnp.int32),
            pltpu.VMEM((CHUNK,), jnp.float32),
            pltpu.SemaphoreType.DMA,
        ],
    )
    return f(dst3, init2)


def _prop_body(enc, za_hbm, zb_hbm, eidx_hbm, outa_hbm, outb_hbm,
               acc, ib_a, ib_b, rows_a, rows_b, isem_a, isem_b, sem_a, sem_b):
    c = lax.axis_index("c")
    s = lax.axis_index("s")
    row0 = s * ROWS_PER_TILE
    nw = 32 if enc else 16
    n_chunks = E_PAD // nw // CHUNK

    for g in range(G):
        @pl.when(c == 0)
        def _():
            pltpu.sync_copy(za_hbm.at[g].at[pl.ds(row0, ROWS_PER_TILE)],
                            acc.at[pl.ds(row0, ROWS_PER_TILE)])

        @pl.when(c == 1)
        def _():
            pltpu.sync_copy(zb_hbm.at[g].at[pl.ds(row0, ROWS_PER_TILE)],
                            acc.at[pl.ds(row0, ROWS_PER_TILE)])

        plsc.subcore_barrier()

        if enc:
            co = E_PAD // 256
        else:
            co = 0

        def run_edges(zref, chunk0, g=g):
            def iload(j, buf, sem):
                pltpu.async_copy(eidx_hbm.at[g].at[chunk0 + j], buf, sem)

            def iwait(j, buf, sem):
                pltpu.make_async_copy(eidx_hbm.at[g].at[chunk0 + j],
                                      buf, sem).wait()

            def gather(j_buf, buf, sem):
                pltpu.async_copy(zref.at[g].at[j_buf.at[0]], buf, sem)

            def gwait(j_buf, buf, sem):
                pltpu.make_async_copy(zref.at[g].at[j_buf.at[0]],
                                      buf, sem).wait()

            def scat(j_buf, buf):
                pltpu.sync_copy(buf, acc.at[j_buf.at[1]], add=True)

            iload(0, ib_a, isem_a)
            iload(1, ib_b, isem_b)
            iwait(0, ib_a, isem_a)
            gather(ib_a, rows_a, sem_a)

            def body(jj, carry):
                j0 = 2 * jj
                iwait(j0 + 1, ib_b, isem_b)
                gather(ib_b, rows_b, sem_b)
                gwait(ib_a, rows_a, sem_a)
                scat(ib_a, rows_a)
                iload(j0 + 2, ib_a, isem_a)
                iwait(j0 + 2, ib_a, isem_a)
                gather(ib_a, rows_a, sem_a)
                gwait(ib_b, rows_b, sem_b)
                scat(ib_b, rows_b)
                iload(j0 + 3, ib_b, isem_b)
                return carry

            lax.fori_loop(0, n_chunks // 2 - 1, body, 0)
            iwait(n_chunks - 1, ib_b, isem_b)
            gather(ib_b, rows_b, sem_b)
            gwait(ib_a, rows_a, sem_a)
            scat(ib_a, rows_a)
            gwait(ib_b, rows_b, sem_b)
            scat(ib_b, rows_b)

        if enc:
            @pl.when(c == 0)
            def _():
                run_edges(za_hbm, s * n_chunks)

            @pl.when(c == 1)
            def _():
                run_edges(za_hbm, co + s * n_chunks)
        else:
            @pl.when(c == 0)
            def _():
                run_edges(za_hbm, s * n_chunks)

            @pl.when(c == 1)
            def _():
                run_edges(zb_hbm, s * n_chunks)

        plsc.subcore_barrier()

        @pl.when(c == 0)
        def _():
            pltpu.sync_copy(acc.at[pl.ds(row0, ROWS_PER_TILE)],
                            outa_hbm.at[g].at[pl.ds(row0, ROWS_PER_TILE)])

        @pl.when(c == 1)
        def _():
            pltpu.sync_copy(acc.at[pl.ds(row0, ROWS_PER_TILE)],
                            outb_hbm.at[g].at[pl.ds(row0, ROWS_PER_TILE)])

        plsc.subcore_barrier()


def _prop_call(enc, za, zb, eidx):
    f = pl.kernel(
        functools.partial(_prop_body, enc),
        out_type=[jax.ShapeDtypeStruct((G, NP, D), jnp.float32),
                  jax.ShapeDtypeStruct((G, NP, D), jnp.float32)],
        mesh=_mesh,
        scratch_types=[
            pltpu.VMEM_SHARED((NP, D), jnp.float32),
            pltpu.VMEM((2, CHUNK), jnp.int32),
            pltpu.VMEM((2, CHUNK), jnp.int32),
            pltpu.VMEM((CHUNK, D), jnp.float32),
            pltpu.VMEM((CHUNK, D), jnp.float32),
            pltpu.SemaphoreType.DMA,
            pltpu.SemaphoreType.DMA,
            pltpu.SemaphoreType.DMA,
            pltpu.SemaphoreType.DMA,
        ],
    )
    return f(za, zb, eidx)


def _row_mask(nb):
    rows = nb * 128 + lax.broadcasted_iota(jnp.int32, (128, D), 0)
    return rows < N


def _t0_body(x_ref, w1_ref, b1_ref, wc1_ref, dinv_ref, pre_ref, z1_ref):
    xb = x_ref[0]
    pre = jnp.dot(xb, w1_ref[0], preferred_element_type=jnp.float32) + b1_ref[0]
    pre_ref[0] = pre
    z1_ref[0] = jnp.dot(pre, wc1_ref[...],
                        preferred_element_type=jnp.float32) * dinv_ref[0]


def _t0(xs, W_fc1, b_fc1, Wc1, dinv_b):
    return pl.pallas_call(
        _t0_body,
        grid=(G, NB),
        in_specs=[
            pl.BlockSpec((1, 128, D), lambda i, nb: (i, nb, 0)),
            pl.BlockSpec((1, D, D), lambda i, nb: (i, 0, 0)),
            pl.BlockSpec((1, 1, D), lambda i, nb: (i, 0, 0)),
            pl.BlockSpec((D, D), lambda i, nb: (0, 0)),
            pl.BlockSpec((1, 128, D), lambda i, nb: (i, nb, 0)),
        ],
        out_specs=[
            pl.BlockSpec((1, 128, D), lambda i, nb: (i, nb, 0)),
            pl.BlockSpec((1, 128, D), lambda i, nb: (i, nb, 0)),
        ],
        out_shape=[jax.ShapeDtypeStruct((G, NP, D), jnp.float32),
                   jax.ShapeDtypeStruct((G, NP, D), jnp.float32)],
    )(xs, W_fc1, b_fc1.reshape(G, 1, D), Wc1, dinv_b)


def _t1_body(ra_ref, rb_ref, dinv_ref, b_ref, w_ref, h_ref, z_ref):
    dinv = dinv_ref[0]
    h = (ra_ref[0] + rb_ref[0]) * dinv + b_ref[0]
    h_ref[0] = h
    z_ref[0] = jnp.dot(h, w_ref[...], preferred_element_type=jnp.float32) * dinv


def _t1(ra, rb, dinv_b, b, w):
    return pl.pallas_call(
        _t1_body,
        grid=(G, NB),
        in_specs=[
            pl.BlockSpec((1, 128, D), lambda i, nb: (i, nb, 0)),
            pl.BlockSpec((1, 128, D), lambda i, nb: (i, nb, 0)),
            pl.BlockSpec((1, 128, D), lambda i, nb: (i, nb, 0)),
            pl.BlockSpec((1, 1, D), lambda i, nb: (0, 0, 0)),
            pl.BlockSpec((D, D), lambda i, nb: (0, 0)),
        ],
        out_specs=[
            pl.BlockSpec((1, 128, D), lambda i, nb: (i, nb, 0)),
            pl.BlockSpec((1, 128, D), lambda i, nb: (i, nb, 0)),
        ],
        out_shape=[jax.ShapeDtypeStruct((G, NP, D), jnp.float32),
                   jax.ShapeDtypeStruct((G, NP, D), jnp.float32)],
    )(ra, rb, dinv_b, b.reshape(1, 1, D), w)


def _t2_body(ra_ref, rb_ref, dinv_ref, b_ref, w_ref,
             enc_ref, za_ref, zb_ref, gs_ref):
    nb = pl.program_id(0)
    mask = _row_mask(nb)
    encs = []
    for i in range(G):
        enc_i = (ra_ref[i] + rb_ref[i]) * dinv_ref[i] + b_ref[0]
        encs.append(enc_i)
    tot = encs[0] + encs[1] + encs[2]
    for i in range(G):
        enc_ref[i] = encs[i]
        dinv = dinv_ref[i]
        za_ref[i] = jnp.dot(encs[i], w_ref[...],
                            preferred_element_type=jnp.float32) * dinv
        dec_i = (tot - encs[i]) * 0.5
        zb_ref[i] = jnp.dot(dec_i, w_ref[...],
                            preferred_element_type=jnp.float32) * dinv
        gs_ref[i, 0] = jnp.sum(jnp.where(mask, encs[i], 0.0), axis=0,
                               keepdims=True)


def _t2(ra, rb, dinv_b, b, w):
    return pl.pallas_call(
        _t2_body,
        grid=(NB,),
        in_specs=[
            pl.BlockSpec((G, 128, D), lambda nb: (0, nb, 0)),
            pl.BlockSpec((G, 128, D), lambda nb: (0, nb, 0)),
            pl.BlockSpec((G, 128, D), lambda nb: (0, nb, 0)),
            pl.BlockSpec((1, 1, D), lambda nb: (0, 0, 0)),
            pl.BlockSpec((D, D), lambda nb: (0, 0)),
        ],
        out_specs=[
            pl.BlockSpec((G, 128, D), lambda nb: (0, nb, 0)),
            pl.BlockSpec((G, 128, D), lambda nb: (0, nb, 0)),
            pl.BlockSpec((G, 128, D), lambda nb: (0, nb, 0)),
            pl.BlockSpec((G, 1, 1, D), lambda nb: (0, nb, 0, 0)),
        ],
        out_shape=[jax.ShapeDtypeStruct((G, NP, D), jnp.float32),
                   jax.ShapeDtypeStruct((G, NP, D), jnp.float32),
                   jax.ShapeDtypeStruct((G, NP, D), jnp.float32),
                   jax.ShapeDtypeStruct((G, NB, 1, D), jnp.float32)],
    )(ra, rb, dinv_b, b.reshape(1, 1, D), w)


def _t3_body(ra_ref, rb_ref, conv1_ref, dinv_ref, b_ref, w_ref,
             za2_ref, zb2_ref, r1_ref):
    nb = pl.program_id(1)
    mask = _row_mask(nb)
    dinv = dinv_ref[0]
    a = ra_ref[0] * dinv + b_ref[0]
    b1 = rb_ref[0] * dinv + b_ref[0]
    za2_ref[0] = jnp.dot(a, w_ref[...], preferred_element_type=jnp.float32) * dinv
    zb2_ref[0] = jnp.dot(b1, w_ref[...], preferred_element_type=jnp.float32) * dinv
    dlt = jnp.where(mask, conv1_ref[0] - a, 0.0)
    r1_ref[0, 0] = jnp.sum(dlt * dlt, axis=0, keepdims=True)


def _t3(ra, rb, conv1, dinv_b, b, w):
    return pl.pallas_call(
        _t3_body,
        grid=(G, NB),
        in_specs=[
            pl.BlockSpec((1, 128, D), lambda i, nb: (i, nb, 0)),
            pl.BlockSpec((1, 128, D), lambda i, nb: (i, nb, 0)),
            pl.BlockSpec((1, 128, D), lambda i, nb: (i, nb, 0)),
            pl.BlockSpec((1, 128, D), lambda i, nb: (i, nb, 0)),
            pl.BlockSpec((1, 1, D), lambda i, nb: (0, 0, 0)),
            pl.BlockSpec((D, D), lambda i, nb: (0, 0)),
        ],
        out_specs=[
            pl.BlockSpec((1, 128, D), lambda i, nb: (i, nb, 0)),
            pl.BlockSpec((1, 128, D), lambda i, nb: (i, nb, 0)),
            pl.BlockSpec((1, 1, 1, D), lambda i, nb: (i, nb, 0, 0)),
        ],
        out_shape=[jax.ShapeDtypeStruct((G, NP, D), jnp.float32),
                   jax.ShapeDtypeStruct((G, NP, D), jnp.float32),
                   jax.ShapeDtypeStruct((G, NB, 1, D), jnp.float32)],
    )(ra, rb, conv1, dinv_b, b.reshape(1, 1, D), w)


def _t4_body(ra_ref, rb_ref, pre_ref, dinv_ref, b_ref, w2a_ref, w2b_ref,
             bf_ref, fin_ref, r2_ref):
    nb = pl.program_id(1)
    mask = _row_mask(nb)
    dinv = dinv_ref[0]
    a2 = ra_ref[0] * dinv + b_ref[0]
    b2 = rb_ref[0] * dinv + b_ref[0]
    fin_ref[0] = (jnp.dot(a2, w2a_ref[...], preferred_element_type=jnp.float32)
                  + jnp.dot(b2, w2b_ref[...], preferred_element_type=jnp.float32)
                  + bf_ref[0])
    dlt = jnp.where(mask, pre_ref[0] - a2, 0.0)
    r2_ref[0, 0] = jnp.sum(dlt * dlt, axis=0, keepdims=True)


def _t4(ra, rb, pre, dinv_b, b, w2a, w2b, bf):
    return pl.pallas_call(
        _t4_body,
        grid=(G, NB),
        in_specs=[
            pl.BlockSpec((1, 128, D), lambda i, nb: (i, nb, 0)),
            pl.BlockSpec((1, 128, D), lambda i, nb: (i, nb, 0)),
            pl.BlockSpec((1, 128, D), lambda i, nb: (i, nb, 0)),
            pl.BlockSpec((1, 128, D), lambda i, nb: (i, nb, 0)),
            pl.BlockSpec((1, 1, D), lambda i, nb: (0, 0, 0)),
            pl.BlockSpec((D, D), lambda i, nb: (0, 0)),
            pl.BlockSpec((D, D), lambda i, nb: (0, 0)),
            pl.BlockSpec((1, 1, D), lambda i, nb: (0, 0, 0)),
        ],
        out_specs=[
            pl.BlockSpec((1, 128, D), lambda i, nb: (i, nb, 0)),
            pl.BlockSpec((1, 1, 1, D), lambda i, nb: (i, nb, 0, 0)),
        ],
        out_shape=[jax.ShapeDtypeStruct((G, NP, D), jnp.float32),
                   jax.ShapeDtypeStruct((G, NB, 1, D), jnp.float32)],
    )(ra, rb, pre, dinv_b, b.reshape(1, 1, D), w2a, w2b, bf.reshape(1, 1, D))


def _t5_body(fin_ref, enc_ref, wmix_ref, gv_ref,
             fuse_ref, used_ref, comp_ref, o0_ref):
    nb = pl.program_id(0)
    mask = _row_mask(nb)
    fins = [fin_ref[i] for i in range(G)]
    encs = [enc_ref[i] for i in range(G)]
    fsum = (fins[0] + fins[1] + fins[2]) * (1.0 / 3.0)
    fuse_ref[...] = jax.nn.sigmoid(fsum)
    esum = (encs[0] + encs[1] + encs[2]) * (1.0 / 3.0)
    used_ref[...] = jnp.where(esum > 0, esum,
                              jnp.exp(jnp.minimum(esum, 0.0)) - 1.0)
    cols = []
    for i in range(G):
        ci = []
        for j in range(G):
            sij = jnp.dot(encs[j], wmix_ref[i],
                          preferred_element_type=jnp.float32)
            tij = jnp.sum(sij * gv_ref[i], axis=1, keepdims=True)
            ci.append(jax.nn.sigmoid(tij))
        den = ci[0] + ci[1] + ci[2]
        cols.extend([c / den for c in ci])
    comp_ref[...] = jnp.concatenate(
        cols + [jnp.zeros((128, D - 3 * G), jnp.float32)], axis=1)
    pairs = [(0, 1), (0, 2), (1, 2)]
    for p, (i, j) in enumerate(pairs):
        dlt = jnp.where(mask, fins[i] - fins[j], 0.0)
        o0_ref[p, 0] = jnp.sum(dlt * dlt, axis=0, keepdims=True)


def _t5(fin, enc, Wmix, gvec):
    return pl.pallas_call(
        _t5_body,
        grid=(NB,),
        in_specs=[
            pl.BlockSpec((G, 128, D), lambda nb: (0, nb, 0)),
            pl.BlockSpec((G, 128, D), lambda nb: (0, nb, 0)),
            pl.BlockSpec((G, D, D), lambda nb: (0, 0, 0)),
            pl.BlockSpec((G, 1, D), lambda nb: (0, 0, 0)),
        ],
        out_specs=[
            pl.BlockSpec((128, D), lambda nb: (nb, 0)),
            pl.BlockSpec((128, D), lambda nb: (nb, 0)),
            pl.BlockSpec((128, D), lambda nb: (nb, 0)),
            pl.BlockSpec((G, 1, 1, D), lambda nb: (0, nb, 0, 0)),
        ],
        out_shape=[jax.ShapeDtypeStruct((NP, D), jnp.float32),
                   jax.ShapeDtypeStruct((NP, D), jnp.float32),
                   jax.ShapeDtypeStruct((NP, D), jnp.float32),
                   jax.ShapeDtypeStruct((G, NB, 1, D), jnp.float32)],
    )(fin, enc, Wmix, gvec.reshape(G, 1, D))


def kernel(x0, x1, x2, ei0, ei1, ei2, W_fc1, b_fc1, Wc1, bc1, Wc2, bc2,
           Wd1, bd1, Wd2, bd2, W_fc2, b_fc2, Wmix):
    pad = ((0, N_PAD_ROWS), (0, 0))
    xs = jnp.stack([jnp.pad(x0, pad), jnp.pad(x1, pad), jnp.pad(x2, pad)])
    pad_idx = (N + (jnp.arange(E_PAD - E, dtype=jnp.int32) % N_PAD_ROWS))
    srcs, dsts = [], []
    for ei in (ei0, ei1, ei2):
        srcs.append(jnp.concatenate([ei[0].astype(jnp.int32), pad_idx]))
        dsts.append(jnp.concatenate([ei[1].astype(jnp.int32), pad_idx]))
    src3 = jnp.stack(srcs).reshape(G, E_PAD // CHUNK, CHUNK)
    dst3 = jnp.stack(dsts).reshape(G, E_PAD // CHUNK, CHUNK)
    eidx = jnp.stack([src3, dst3], axis=2)
    init2 = jnp.stack([jnp.ones((NP,), jnp.float32),
                       jnp.zeros((NP,), jnp.float32)]).reshape(2, 1, NP)
    zeros3 = jnp.zeros((G, NP, D), jnp.float32)

    dega, degb = _deg_call(dst3, init2)
    deg = (dega + degb).reshape(G, NP)
    dinv = lax.rsqrt(jnp.maximum(deg, 1e-12))
    dinv_b = jnp.broadcast_to(dinv[:, :, None], (G, NP, D))

    pre, z1 = _t0(xs, W_fc1, b_fc1, Wc1, dinv_b)
    r1a, r1b = _prop_call(True, z1, zeros3, eidx)
    conv1, z2 = _t1(r1a, r1b, dinv_b, bc1, Wc2)
    r2a, r2b = _prop_call(True, z2, zeros3, eidx)
    enc, za, zb, gsum = _t2(r2a, r2b, dinv_b, bc2, Wd1)

    r3a, r3b = _prop_call(False, za, zb, eidx)
    za2, zb2, r1sum = _t3(r3a, r3b, conv1, dinv_b, bd1, Wd2)
    r4a, r4b = _prop_call(False, za2, zb2, eidx)
    fin, r2sum = _t4(r4a, r4b, pre, dinv_b, bd2,
                     W_fc2[:D], W_fc2[D:], b_fc2)

    gvec = jax.nn.sigmoid(jnp.sum(gsum, axis=(1, 2)) / N)
    fuse, used, comp, o0sum = _t5(fin, enc, Wmix, gvec)

    obf1 = 0.0
    for i in range(G):
        obf1 = obf1 + (jnp.sqrt(jnp.sum(r2sum[i]))
                       + jnp.sqrt(jnp.sum(r1sum[i]))) / 2.0
    obf0 = 0.0
    for p in range(3):
        obf0 = obf0 + 2.0 * jnp.sqrt(jnp.sum(o0sum[p]))

    return (fuse[:N], used[:N], comp[:N, :3 * G], obf1, obf0)
